# Initial kernel scaffold; baseline (speedup 1.0000x reference)
import jax
import jax.numpy as jnp
from jax.experimental import pallas as pl


def kernel(indices, table):
    raise NotImplementedError("write your pallas kernel here")


# speedup vs baseline: 1.8732x; 1.8732x over previous
import jax
import jax.numpy as jnp
from jax import lax
from jax.experimental import pallas as pl
from jax.experimental.pallas import tpu as pltpu
from jax.experimental.pallas import tpu_sc as plsc

NUM_ROWS = 1000000
D = 64
B = 16384 * 50
NC, NS, L = 2, 16, 16
NW = NC * NS
BPW = B // NW
CHUNK = 128
NBUF = 8
NCHUNK = BPW // CHUNK
ROUNDS = NCHUNK // NBUF


def _body(idx_hbm, table_hbm, out_hbm, idx_v, *rest):
    bufs = rest[:NBUF]
    gsems = rest[NBUF:2 * NBUF]
    ssems = rest[2 * NBUF:3 * NBUF]

    wid = lax.axis_index("s") * NC + lax.axis_index("c")
    base = wid * BPW

    pltpu.sync_copy(idx_hbm.at[pl.ds(base, BPW)], idx_v)

    def gather(i, b):
        return pltpu.make_async_copy(
            table_hbm.at[idx_v.at[pl.ds(i * CHUNK, CHUNK)]], bufs[b], gsems[b])

    def store(i, b):
        return pltpu.make_async_copy(
            bufs[b], out_hbm.at[pl.ds(base + i * CHUNK, CHUNK)], ssems[b])

    def fix_padding(i, b):
        cbase = i * CHUNK
        vregs = [idx_v[pl.ds(cbase + j * L, L)] for j in range(CHUNK // L)]
        acc = vregs[0] == 0
        for v in vregs[1:]:
            acc = acc | (v == 0)
        cnt = plsc.all_reduce_population_count(acc)

        def do_fix(b=b, vregs=vregs):
            zeros = jnp.zeros((L,), jnp.float32)
            for j, v in enumerate(vregs):
                m = v == 0
                rows = lax.iota(jnp.int32, L) + (j * L)

                def col_body(c, carry, m=m, rows=rows):
                    cols = jnp.full((L,), c, jnp.int32)
                    plsc.store_scatter(bufs[b], [rows, cols], zeros, mask=m)
                    return carry

                lax.fori_loop(0, D, col_body, 0)

        lax.cond(cnt[0] > 0, do_fix, lambda: None)

    def run_round(r, issue_next):
        for b in range(NBUF):
            i = r * NBUF + b
            gather(i, b).wait()
            fix_padding(i, b)
            store(i, b).start()
        if issue_next:
            for b in range(NBUF):
                i = r * NBUF + b
                store(i, b).wait()
                gather(i + NBUF, b).start()

    for b in range(NBUF):
        gather(b, b).start()

    def loop_body(r, carry):
        run_round(r, issue_next=True)
        return carry

    lax.fori_loop(0, ROUNDS - 1, loop_body, 0)

    run_round(ROUNDS - 1, issue_next=False)
    for b in range(NBUF):
        i = (ROUNDS - 1) * NBUF + b
        store(i, b).wait()


_run = pl.kernel(
    _body,
    out_type=jax.ShapeDtypeStruct((B, D), jnp.float32),
    mesh=plsc.VectorSubcoreMesh(core_axis_name="c", subcore_axis_name="s"),
    compiler_params=pltpu.CompilerParams(
        needs_layout_passes=False, use_tc_tiling_on_sc=False),
    scratch_types=(
        [pltpu.VMEM((BPW,), jnp.int32)]
        + [pltpu.VMEM((CHUNK, D), jnp.float32) for _ in range(NBUF)]
        + [pltpu.SemaphoreType.DMA for _ in range(2 * NBUF)]
    ),
)


def kernel(indices, table):
    assert indices.shape == (16384, 50) and table.shape == (NUM_ROWS, D)
    idx = indices.reshape(-1).astype(jnp.int32)
    out = _run(idx, table)
    return out.reshape(indices.shape[0], indices.shape[1], D)
